# baseline (speedup 1.0000x reference)
import jax
import jax.numpy as jnp
from jax.experimental import pallas as pl

EMB = 128
L = 3
G_MAX = 128
ATOM_F = 9
BOND_F = 3


def _bn(h, g, b):
    return g * h / jnp.sqrt(1.0 + 1e-5) + b


def _copy_kernel(x_ref, o_ref):
    o_ref[...] = x_ref[...]


def _pallas_copy(x):
    return pl.pallas_call(
        _copy_kernel,
        out_shape=jax.ShapeDtypeStruct(x.shape, x.dtype),
    )(x)


def kernel(x, edge_index, edge_attr, batch, params):
    num_graphs = G_MAX
    src, dst = edge_index[0], edge_index[1]
    n_nodes = x.shape[0]
    h0 = jnp.zeros((n_nodes, EMB), dtype=jnp.float32)
    for f in range(ATOM_F):
        h0 = h0 + params['atom_emb'][f][x[:, f]]
    vn = jnp.broadcast_to(params['vn_emb'][0], (num_graphs, EMB))
    h_list = [h0]
    fp = []
    for layer in range(L):
        h_in = h_list[layer] + vn[batch]
        ee = jnp.zeros((edge_attr.shape[0], EMB), dtype=jnp.float32)
        for f in range(BOND_F):
            ee = ee + params['bond_emb'][layer, f][edge_attr[:, f]]
        msg = jax.nn.relu(h_in[src] + ee)
        aggr = jax.ops.segment_sum(msg, dst, num_segments=n_nodes)
        t = (1.0 + params['eps'][layer]) * h_in + aggr
        t = t @ params['mlp_W1'][layer].T + params['mlp_b1'][layer]
        t = _bn(t, params['mlp_g1'][layer], params['mlp_be1'][layer])
        t = jax.nn.relu(t)
        t = t @ params['mlp_W2'][layer].T + params['mlp_b2'][layer]
        h = _bn(t, params['bn_g'][layer], params['bn_b'][layer])
        if layer != L - 1:
            h = jax.nn.relu(h)
        h_list.append(h)
        if layer < L - 1:
            vt = jax.ops.segment_sum(h_in, batch, num_segments=num_graphs) + vn
            v = vt @ params['vn_W1'][layer].T + params['vn_b1'][layer]
            v = jax.nn.relu(_bn(v, params['vn_g1'][layer], params['vn_be1'][layer]))
            v = v @ params['vn_W2'][layer].T + params['vn_b2'][layer]
            v = jax.nn.relu(_bn(v, params['vn_g2'][layer], params['vn_be2'][layer]))
            vn = v
            fp.append(vn)
    return _pallas_copy(h_list[-1]), jnp.stack(fp, axis=1)


# speedup vs baseline: 2.5782x; 2.5782x over previous
import functools

import jax
import jax.numpy as jnp
from jax import lax
from jax.experimental import pallas as pl
from jax.experimental.pallas import tpu as pltpu
from jax.experimental.pallas import tpu_sc as plsc

N = 10000
E = 320000
EMB = 128
G = 128
NC = 2
NS = 16
EPT = E // NS
C = 80
NCHUNK = EPT // C
SB = 50
NSB = NCHUNK // SB
HALF = N // NC
PAD = 64
AROWS = HALF + PAD
BN = 2000
NBLK = N // BN
_BN_S = 1.0 / (1.0 + 1e-5) ** 0.5


def _edge_body(hin, src4, ce4, dst5, comb, out,
               src_v, ce_v, dst_v, comb_v, rows0, rows1, accum, gsem0, gsem1):
    cid = lax.axis_index("c")
    sid = lax.axis_index("s")

    pltpu.sync_copy(comb, comb_v)

    zero16 = jnp.zeros((16,), jnp.float32)

    def zrow(r, _):
        for c in range(8):
            rows0[r, pl.ds(c * 16, 16)] = zero16
        return 0
    lax.fori_loop(0, C, zrow, 0)

    @pl.when(sid < NS - 1)
    def _():
        for k in range(4):
            pltpu.sync_copy(rows0, accum.at[pl.ds(sid * 320 + k * C, C)])

    @pl.when(sid == NS - 1)
    def _():
        for k in range(3):
            pltpu.sync_copy(rows0, accum.at[pl.ds(4800 + k * C, C)])
        pltpu.sync_copy(rows0.at[pl.ds(0, 24)], accum.at[pl.ds(5040, 24)])
    plsc.subcore_barrier()

    def process(j, rows, gsem):
        pltpu.make_async_copy(hin.at[src_v.at[0]], rows, gsem).wait()

        def group_body(g, _):
            gb = g * 16
            cev = ce_v[j, pl.ds(gb, 16)] * EMB
            for k in range(16):
                ce = cev[k]
                e = gb + k
                for c in range(8):
                    rv = rows[e, pl.ds(c * 16, 16)]
                    ev = comb_v[pl.ds(ce + c * 16, 16)]
                    rows[e, pl.ds(c * 16, 16)] = jnp.maximum(rv + ev, 0.0)
            return 0
        lax.fori_loop(0, C // 16, group_body, 0)
        pltpu.sync_copy(rows, accum.at[dst_v.at[j]], add=True)

    def sb_body(sb, _):
        pltpu.sync_copy(src4.at[sid].at[sb], src_v)
        pltpu.sync_copy(ce4.at[sid].at[sb], ce_v)
        pltpu.sync_copy(dst5.at[cid].at[sid].at[sb], dst_v)
        pltpu.async_copy(hin.at[src_v.at[0]], rows0, gsem0)

        def pair_body(t, _):
            j0 = 2 * t
            j1 = j0 + 1

            pltpu.async_copy(hin.at[src_v.at[j1]], rows1, gsem1)
            process(j0, rows0, gsem0)

            @pl.when(j0 + 2 < SB)
            def _():
                pltpu.async_copy(hin.at[src_v.at[j0 + 2]], rows0, gsem0)

            process(j1, rows1, gsem1)
            return 0

        lax.fori_loop(0, SB // 2, pair_body, 0)
        return 0

    lax.fori_loop(0, NSB, sb_body, 0)

    plsc.subcore_barrier()
    @pl.when(sid < NS - 1)
    def _():
        pltpu.sync_copy(accum.at[pl.ds(sid * 312, 312)],
                        out.at[cid].at[pl.ds(sid * 312, 312)])

    @pl.when(sid == NS - 1)
    def _():
        pltpu.sync_copy(accum.at[pl.ds(4680, 320)],
                        out.at[cid].at[pl.ds(4680, 320)])


_edge_kernel = functools.partial(
    pl.kernel,
    out_type=jax.ShapeDtypeStruct((NC, HALF, EMB), jnp.float32),
    mesh=plsc.VectorSubcoreMesh(core_axis_name="c", subcore_axis_name="s"),
    scratch_types=[
        pltpu.VMEM((SB, C), jnp.int32),
        pltpu.VMEM((SB, C), jnp.int32),
        pltpu.VMEM((SB, C), jnp.int32),
        pltpu.VMEM((216 * EMB,), jnp.float32),
        pltpu.VMEM((C, EMB), jnp.float32),
        pltpu.VMEM((C, EMB), jnp.float32),
        pltpu.VMEM_SHARED((AROWS, EMB), jnp.float32),
        pltpu.SemaphoreType.DMA,
        pltpu.SemaphoreType.DMA,
    ],
)(_edge_body)


def _prep_body(pk_ref, ce_ref, da_ref, db_ref):
    a = pk_ref[0, 0]
    b = pk_ref[0, 1]
    c = pk_ref[0, 2]
    d = pk_ref[0, 3]
    ce_ref[0, 0] = a * 36 + b * 6 + c
    pad = HALF + lax.broadcasted_iota(jnp.int32, (EPT,), 0) % PAD
    da_ref[0, 0] = jnp.where(d < HALF, d, pad)
    db_ref[0, 0] = jnp.where(d >= HALF, d - HALF, pad)


def _k_prep(packed):
    spec = pl.BlockSpec((1, 1, EPT), lambda i: (i, 0, 0))
    shp = jax.ShapeDtypeStruct((NS, 1, EPT), jnp.int32)
    return pl.pallas_call(
        _prep_body,
        grid=(NS,),
        in_specs=[pl.BlockSpec((1, 8, EPT), lambda i: (i, 0, 0))],
        out_specs=[spec, spec, spec],
        out_shape=[shp, shp, shp],
    )(packed)


def _comb_body(be_ref, out_ref):
    b2 = be_ref[0, 2]
    for a in range(6):
        for b in range(6):
            row = be_ref[0, 0, a][None, :] + be_ref[0, 1, b][None, :]
            out_ref[0, pl.ds((a * 6 + b) * 6, 6), :] = row + b2


def _k_comb(bond_emb):
    return pl.pallas_call(
        _comb_body,
        grid=(3,),
        in_specs=[pl.BlockSpec((1, 3, 6, EMB), lambda i: (i, 0, 0, 0))],
        out_specs=pl.BlockSpec((1, 216, EMB), lambda i: (i, 0, 0)),
        out_shape=jax.ShapeDtypeStruct((3, 216, EMB), jnp.float32),
    )(bond_emb)


def _onehot(v, n):
    return (v[:, None] == lax.broadcasted_iota(jnp.int32, (v.shape[0], n), 1)
            ).astype(jnp.float32)


def _atom_body(xt_ref, batch_ref, emb_ref, vne_ref, hin_ref, pool_ref):
    i = pl.program_id(0)
    h = jnp.zeros((BN, EMB), jnp.float32)
    for f in range(9):
        oh = _onehot(xt_ref[0, f, :], 128)
        h = h + lax.dot_general(oh, emb_ref[f], (((1,), (0,)), ((), ())),
                                preferred_element_type=jnp.float32)
    h = h + vne_ref[0, :][None, :]
    hin_ref[...] = h
    ohg = _onehot(batch_ref[0, 0, :], G)
    p = lax.dot_general(ohg, h, (((0,), (0,)), ((), ())),
                        preferred_element_type=jnp.float32)

    @pl.when(i == 0)
    def _():
        pool_ref[...] = p

    @pl.when(i != 0)
    def _():
        pool_ref[...] += p


def _k_atom(xt, batch3, emb_pad, vne):
    return pl.pallas_call(
        _atom_body,
        grid=(NBLK,),
        in_specs=[
            pl.BlockSpec((1, 16, BN), lambda i: (i, 0, 0)),
            pl.BlockSpec((1, 1, BN), lambda i: (i, 0, 0)),
            pl.BlockSpec((9, 128, EMB), lambda i: (0, 0, 0)),
            pl.BlockSpec((1, EMB), lambda i: (0, 0)),
        ],
        out_specs=[
            pl.BlockSpec((BN, EMB), lambda i: (i, 0)),
            pl.BlockSpec((G, EMB), lambda i: (0, 0)),
        ],
        out_shape=[
            jax.ShapeDtypeStruct((N, EMB), jnp.float32),
            jax.ShapeDtypeStruct((G, EMB), jnp.float32),
        ],
    )(xt, batch3, emb_pad, vne)


def _vn_body(pool_ref, vnp_ref, w1_ref, b1_ref, g1_ref, be1_ref,
             w2_ref, b2_ref, g2_ref, be2_ref, out_ref):
    vt = pool_ref[...] + vnp_ref[...]
    v = lax.dot_general(vt, w1_ref[...], (((1,), (1,)), ((), ())),
                        preferred_element_type=jnp.float32) + b1_ref[0][None, :]
    v = jnp.maximum(v * (g1_ref[0] * _BN_S)[None, :] + be1_ref[0][None, :], 0.0)
    v = lax.dot_general(v, w2_ref[...], (((1,), (1,)), ((), ())),
                        preferred_element_type=jnp.float32) + b2_ref[0][None, :]
    v = jnp.maximum(v * (g2_ref[0] * _BN_S)[None, :] + be2_ref[0][None, :], 0.0)
    out_ref[...] = v


def _k_vn(pool, vn_prev, w1, b1, g1, be1, w2, b2, g2, be2):
    vecs = [v.reshape(1, EMB) for v in (b1, g1, be1, b2, g2, be2)]
    return pl.pallas_call(
        _vn_body,
        out_shape=jax.ShapeDtypeStruct((G, EMB), jnp.float32),
    )(pool, vn_prev, w1, *vecs[:3], w2, *vecs[3:])


def _mlp_body(layer, hin_ref, a0_ref, batch_ref, vnn_ref, eps_ref,
              w1_ref, b1_ref, g1_ref, be1_ref, w2_ref, b2_ref, bg_ref, bb_ref,
              hout_ref, pool_ref):
    i = pl.program_id(0)
    t = (1.0 + eps_ref[0, 0]) * hin_ref[...] + a0_ref[...]
    t = lax.dot_general(t, w1_ref[...], (((1,), (1,)), ((), ())),
                        preferred_element_type=jnp.float32) + b1_ref[0][None, :]
    t = jnp.maximum(t * (g1_ref[0] * _BN_S)[None, :] + be1_ref[0][None, :], 0.0)
    t = lax.dot_general(t, w2_ref[...], (((1,), (1,)), ((), ())),
                        preferred_element_type=jnp.float32) + b2_ref[0][None, :]
    h = t * (bg_ref[0] * _BN_S)[None, :] + bb_ref[0][None, :]
    if layer < 2:
        h = jnp.maximum(h, 0.0)
        oh = _onehot(batch_ref[0, 0, :], G)
        h = h + lax.dot_general(oh, vnn_ref[...], (((1,), (0,)), ((), ())),
                                preferred_element_type=jnp.float32)
    hout_ref[...] = h
    if layer == 0:
        p = lax.dot_general(oh, h, (((0,), (0,)), ((), ())),
                            preferred_element_type=jnp.float32)

        @pl.when(i == 0)
        def _():
            pool_ref[...] = p

        @pl.when(i != 0)
        def _():
            pool_ref[...] += p


def _k_mlp(layer, hin, a0, batch3, vnn, eps, w1, b1, g1, be1, w2, b2, bg, bb):
    body = functools.partial(_mlp_body, layer)
    nblock = pl.BlockSpec((BN, EMB), lambda i: (i, 0))
    gblock = pl.BlockSpec((G, EMB), lambda i: (0, 0))
    vblock = pl.BlockSpec((1, EMB), lambda i: (0, 0))
    wblock = pl.BlockSpec((EMB, EMB), lambda i: (0, 0))
    vecs = [v.reshape(1, EMB) for v in (b1, g1, be1, b2, bg, bb)]
    return pl.pallas_call(
        body,
        grid=(NBLK,),
        in_specs=[
            nblock, nblock,
            pl.BlockSpec((1, 1, BN), lambda i: (i, 0, 0)),
            gblock,
            pl.BlockSpec((1, 1), lambda i: (0, 0)),
            wblock, vblock, vblock, vblock,
            wblock, vblock, vblock, vblock,
        ],
        out_specs=[nblock, gblock],
        out_shape=[jax.ShapeDtypeStruct((N, EMB), jnp.float32),
                   jax.ShapeDtypeStruct((G, EMB), jnp.float32)],
    )(hin, a0, batch3, vnn, eps, w1, *vecs[:3], w2, *vecs[3:])


def kernel(x, edge_index, edge_attr, batch, params):
    p = params
    xp = jnp.pad(x.astype(jnp.int32), ((0, 0), (0, 7)))
    xt = xp.reshape(NBLK, BN, 16).transpose(0, 2, 1)
    batch3 = batch.astype(jnp.int32).reshape(NBLK, 1, BN)
    ea = edge_attr.astype(jnp.int32)
    packed = jnp.stack([ea[:, 0], ea[:, 1], ea[:, 2],
                        edge_index[1].astype(jnp.int32)])
    packed = jnp.pad(packed, ((0, 4), (0, 0)))
    packed = packed.reshape(8, NS, EPT).transpose(1, 0, 2)
    src3 = edge_index[0].astype(jnp.int32).reshape(NS, NSB, SB, C)
    emb_pad = jnp.pad(p['atom_emb'], ((0, 0), (0, 28), (0, 0)))

    ce3, da3, db3 = _k_prep(packed)
    ce3 = ce3.reshape(NS, NSB, SB, C)
    dst4 = jnp.stack([da3.reshape(NS, NSB, SB, C),
                      db3.reshape(NS, NSB, SB, C)])

    comb_all = _k_comb(p['bond_emb']).reshape(3, 216 * EMB)

    hin0, pool0 = _k_atom(xt, batch3, emb_pad, p['vn_emb'])
    vn0 = jnp.broadcast_to(p['vn_emb'][0][None, :], (G, EMB))

    def vn_step(layer, pool, vn_prev):
        return _k_vn(pool, vn_prev,
                     p['vn_W1'][layer], p['vn_b1'][layer], p['vn_g1'][layer],
                     p['vn_be1'][layer], p['vn_W2'][layer], p['vn_b2'][layer],
                     p['vn_g2'][layer], p['vn_be2'][layer])

    def mlp_args(layer):
        return (p['eps'][layer].reshape(1, 1), p['mlp_W1'][layer],
                p['mlp_b1'][layer], p['mlp_g1'][layer], p['mlp_be1'][layer],
                p['mlp_W2'][layer], p['mlp_b2'][layer], p['bn_g'][layer],
                p['bn_b'][layer])

    vn1 = vn_step(0, pool0, vn0)
    aggr = _edge_kernel(hin0, src3, ce3, dst4, comb_all[0]).reshape(N, EMB)
    hin1, pool1 = _k_mlp(0, hin0, aggr, batch3, vn1, *mlp_args(0))

    vn2 = vn_step(1, pool1, vn1)
    aggr = _edge_kernel(hin1, src3, ce3, dst4, comb_all[1]).reshape(N, EMB)
    hin2, _ = _k_mlp(1, hin1, aggr, batch3, vn2, *mlp_args(1))

    aggr = _edge_kernel(hin2, src3, ce3, dst4, comb_all[2]).reshape(N, EMB)
    h_out, _ = _k_mlp(2, hin2, aggr, batch3, vn2, *mlp_args(2))

    return h_out, jnp.stack([vn1, vn2], axis=1)


# speedup vs baseline: 4.6409x; 1.8000x over previous
import functools

import jax
import jax.numpy as jnp
from jax import lax
from jax.experimental import pallas as pl
from jax.experimental.pallas import tpu as pltpu
from jax.experimental.pallas import tpu_sc as plsc

N = 10000
E = 320000
EMB = 128
G = 128
NC = 2
NS = 16
EPT = E // (NC * NS)
C = 80
SB = 25
NSB = EPT // (C * SB)
TAB = 48
BN = 2000
NBLK = N // BN
_BN_S = 1.0 / (1.0 + 1e-5) ** 0.5


def _edge_body(hin, src5, ce5, cc5, dst5, comb, out,
               src_v, ce_v, cc_v, dst_v, tab_v, rows0, rows1, accum,
               gsem0, gsem1):
    cid = lax.axis_index("c")
    sid = lax.axis_index("s")

    pltpu.sync_copy(comb, tab_v)

    zero16 = jnp.zeros((16,), jnp.float32)

    def zrow(r, _):
        for c in range(8):
            rows0[r, pl.ds(c * 16, 16)] = zero16
        return 0
    lax.fori_loop(0, C, zrow, 0)

    @pl.when(sid < NS - 1)
    def _():
        for k in range(8):
            pltpu.sync_copy(rows0, accum.at[pl.ds(sid * 640 + k * C, C)])

    @pl.when(sid == NS - 1)
    def _():
        for k in range(5):
            pltpu.sync_copy(rows0, accum.at[pl.ds(9600 + k * C, C)])
    plsc.subcore_barrier()

    def process(j, rows, gsem):
        pltpu.make_async_copy(hin.at[src_v.at[0]], rows, gsem).wait()

        def group_body(g, _):
            gb = g * 16
            cev = ce_v[j, pl.ds(gb, 16)]
            ccv = cc_v[j, pl.ds(gb, 16)]
            for k in range(16):
                ce = cev[k]
                cc = ccv[k]
                e = gb + k
                for c in range(8):
                    rv = rows[e, pl.ds(c * 16, 16)]
                    ev = tab_v[pl.ds(ce + c * 16, 16)]
                    e2 = tab_v[pl.ds(cc + c * 16, 16)]
                    rows[e, pl.ds(c * 16, 16)] = jnp.maximum(rv + ev + e2, 0.0)
            return 0
        lax.fori_loop(0, C // 16, group_body, 0)
        pltpu.sync_copy(rows, accum.at[dst_v.at[j]], add=True)

    def sb_body(sb, _):
        pltpu.sync_copy(src5.at[cid].at[sid].at[sb], src_v)
        pltpu.sync_copy(ce5.at[cid].at[sid].at[sb], ce_v)
        pltpu.sync_copy(cc5.at[cid].at[sid].at[sb], cc_v)
        pltpu.sync_copy(dst5.at[cid].at[sid].at[sb], dst_v)
        pltpu.async_copy(hin.at[src_v.at[0]], rows0, gsem0)

        def pair_body(t, _):
            j0 = 2 * t
            j1 = j0 + 1
            pltpu.async_copy(hin.at[src_v.at[j1]], rows1, gsem1)
            process(j0, rows0, gsem0)
            pltpu.async_copy(hin.at[src_v.at[j0 + 2]], rows0, gsem0)
            process(j1, rows1, gsem1)
            return 0

        lax.fori_loop(0, SB // 2, pair_body, 0)
        process(SB - 1, rows0, gsem0)
        return 0

    lax.fori_loop(0, NSB, sb_body, 0)

    plsc.subcore_barrier()
    @pl.when(sid < NS - 1)
    def _():
        pltpu.sync_copy(accum.at[pl.ds(sid * 640, 640)],
                        out.at[cid].at[pl.ds(sid * 640, 640)])

    @pl.when(sid == NS - 1)
    def _():
        pltpu.sync_copy(accum.at[pl.ds(9600, 400)],
                        out.at[cid].at[pl.ds(9600, 400)])


_edge_kernel = functools.partial(
    pl.kernel,
    out_type=jax.ShapeDtypeStruct((NC, N, EMB), jnp.float32),
    mesh=plsc.VectorSubcoreMesh(core_axis_name="c", subcore_axis_name="s"),
    scratch_types=[
        pltpu.VMEM((SB, C), jnp.int32),
        pltpu.VMEM((SB, C), jnp.int32),
        pltpu.VMEM((SB, C), jnp.int32),
        pltpu.VMEM((SB, C), jnp.int32),
        pltpu.VMEM((TAB * EMB,), jnp.float32),
        pltpu.VMEM((C, EMB), jnp.float32),
        pltpu.VMEM((C, EMB), jnp.float32),
        pltpu.VMEM_SHARED((N, EMB), jnp.float32),
        pltpu.SemaphoreType.DMA,
        pltpu.SemaphoreType.DMA,
    ],
)(_edge_body)


def _prep_body(pk_ref, ce_ref, cc_ref):
    a = pk_ref[0, 0]
    b = pk_ref[0, 1]
    c = pk_ref[0, 2]
    ce_ref[0, 0] = (a * 6 + b) * EMB
    cc_ref[0, 0] = (c + 36) * EMB


def _k_prep(packed):
    nt = NC * NS
    spec = pl.BlockSpec((1, 1, EPT), lambda i: (i, 0, 0))
    shp = jax.ShapeDtypeStruct((nt, 1, EPT), jnp.int32)
    return pl.pallas_call(
        _prep_body,
        grid=(nt,),
        in_specs=[pl.BlockSpec((1, 8, EPT), lambda i: (i, 0, 0))],
        out_specs=[spec, spec],
        out_shape=[shp, shp],
    )(packed)


def _comb_body(be_ref, out_ref):
    out_ref[0, :, :] = jnp.zeros((TAB, EMB), jnp.float32)
    for a in range(6):
        out_ref[0, pl.ds(a * 6, 6), :] = be_ref[0, 0, a][None, :] + be_ref[0, 1]
    out_ref[0, pl.ds(36, 6), :] = be_ref[0, 2]


def _k_comb(bond_emb):
    return pl.pallas_call(
        _comb_body,
        grid=(3,),
        in_specs=[pl.BlockSpec((1, 3, 6, EMB), lambda i: (i, 0, 0, 0))],
        out_specs=pl.BlockSpec((1, TAB, EMB), lambda i: (i, 0, 0)),
        out_shape=jax.ShapeDtypeStruct((3, TAB, EMB), jnp.float32),
    )(bond_emb)


def _onehot(v, n):
    return (v[:, None] == lax.broadcasted_iota(jnp.int32, (v.shape[0], n), 1)
            ).astype(jnp.float32)


def _atom_body(xt_ref, batch_ref, emb_ref, vne_ref, hin_ref, pool_ref):
    i = pl.program_id(0)
    h = jnp.zeros((BN, EMB), jnp.float32)
    for f in range(9):
        oh = _onehot(xt_ref[0, f, :], 128)
        h = h + lax.dot_general(oh, emb_ref[f], (((1,), (0,)), ((), ())),
                                preferred_element_type=jnp.float32)
    h = h + vne_ref[0, :][None, :]
    hin_ref[...] = h
    ohg = _onehot(batch_ref[0, 0, :], G)
    p = lax.dot_general(ohg, h, (((0,), (0,)), ((), ())),
                        preferred_element_type=jnp.float32)

    @pl.when(i == 0)
    def _():
        pool_ref[...] = p

    @pl.when(i != 0)
    def _():
        pool_ref[...] += p


def _k_atom(xt, batch3, emb_pad, vne):
    return pl.pallas_call(
        _atom_body,
        grid=(NBLK,),
        in_specs=[
            pl.BlockSpec((1, 16, BN), lambda i: (i, 0, 0)),
            pl.BlockSpec((1, 1, BN), lambda i: (i, 0, 0)),
            pl.BlockSpec((9, 128, EMB), lambda i: (0, 0, 0)),
            pl.BlockSpec((1, EMB), lambda i: (0, 0)),
        ],
        out_specs=[
            pl.BlockSpec((BN, EMB), lambda i: (i, 0)),
            pl.BlockSpec((G, EMB), lambda i: (0, 0)),
        ],
        out_shape=[
            jax.ShapeDtypeStruct((N, EMB), jnp.float32),
            jax.ShapeDtypeStruct((G, EMB), jnp.float32),
        ],
    )(xt, batch3, emb_pad, vne)


def _vn_body(pool_ref, vnp_ref, w1_ref, b1_ref, g1_ref, be1_ref,
             w2_ref, b2_ref, g2_ref, be2_ref, out_ref):
    vt = pool_ref[...] + vnp_ref[...]
    v = lax.dot_general(vt, w1_ref[...], (((1,), (1,)), ((), ())),
                        preferred_element_type=jnp.float32) + b1_ref[0][None, :]
    v = jnp.maximum(v * (g1_ref[0] * _BN_S)[None, :] + be1_ref[0][None, :], 0.0)
    v = lax.dot_general(v, w2_ref[...], (((1,), (1,)), ((), ())),
                        preferred_element_type=jnp.float32) + b2_ref[0][None, :]
    v = jnp.maximum(v * (g2_ref[0] * _BN_S)[None, :] + be2_ref[0][None, :], 0.0)
    out_ref[...] = v


def _k_vn(pool, vn_prev, w1, b1, g1, be1, w2, b2, g2, be2):
    vecs = [v.reshape(1, EMB) for v in (b1, g1, be1, b2, g2, be2)]
    return pl.pallas_call(
        _vn_body,
        out_shape=jax.ShapeDtypeStruct((G, EMB), jnp.float32),
    )(pool, vn_prev, w1, *vecs[:3], w2, *vecs[3:])


def _mlp_body(layer, hin_ref, a0_ref, a1_ref, batch_ref, vnn_ref, eps_ref,
              w1_ref, b1_ref, g1_ref, be1_ref, w2_ref, b2_ref, bg_ref, bb_ref,
              hout_ref, pool_ref):
    i = pl.program_id(0)
    t = (1.0 + eps_ref[0, 0]) * hin_ref[...] + (a0_ref[...] + a1_ref[...])
    t = lax.dot_general(t, w1_ref[...], (((1,), (1,)), ((), ())),
                        preferred_element_type=jnp.float32) + b1_ref[0][None, :]
    t = jnp.maximum(t * (g1_ref[0] * _BN_S)[None, :] + be1_ref[0][None, :], 0.0)
    t = lax.dot_general(t, w2_ref[...], (((1,), (1,)), ((), ())),
                        preferred_element_type=jnp.float32) + b2_ref[0][None, :]
    h = t * (bg_ref[0] * _BN_S)[None, :] + bb_ref[0][None, :]
    if layer < 2:
        h = jnp.maximum(h, 0.0)
        oh = _onehot(batch_ref[0, 0, :], G)
        h = h + lax.dot_general(oh, vnn_ref[...], (((1,), (0,)), ((), ())),
                                preferred_element_type=jnp.float32)
    hout_ref[...] = h
    if layer == 0:
        p = lax.dot_general(oh, h, (((0,), (0,)), ((), ())),
                            preferred_element_type=jnp.float32)

        @pl.when(i == 0)
        def _():
            pool_ref[...] = p

        @pl.when(i != 0)
        def _():
            pool_ref[...] += p


def _k_mlp(layer, hin, a0, a1, batch3, vnn, eps, w1, b1, g1, be1, w2, b2, bg, bb):
    body = functools.partial(_mlp_body, layer)
    nblock = pl.BlockSpec((BN, EMB), lambda i: (i, 0))
    gblock = pl.BlockSpec((G, EMB), lambda i: (0, 0))
    vblock = pl.BlockSpec((1, EMB), lambda i: (0, 0))
    wblock = pl.BlockSpec((EMB, EMB), lambda i: (0, 0))
    vecs = [v.reshape(1, EMB) for v in (b1, g1, be1, b2, bg, bb)]
    return pl.pallas_call(
        body,
        grid=(NBLK,),
        in_specs=[
            nblock, nblock, nblock,
            pl.BlockSpec((1, 1, BN), lambda i: (i, 0, 0)),
            gblock,
            pl.BlockSpec((1, 1), lambda i: (0, 0)),
            wblock, vblock, vblock, vblock,
            wblock, vblock, vblock, vblock,
        ],
        out_specs=[nblock, gblock],
        out_shape=[jax.ShapeDtypeStruct((N, EMB), jnp.float32),
                   jax.ShapeDtypeStruct((G, EMB), jnp.float32)],
    )(hin, a0, a1, batch3, vnn, eps, w1, *vecs[:3], w2, *vecs[3:])


def kernel(x, edge_index, edge_attr, batch, params):
    p = params
    xp = jnp.pad(x.astype(jnp.int32), ((0, 0), (0, 7)))
    xt = xp.reshape(NBLK, BN, 16).transpose(0, 2, 1)
    batch3 = batch.astype(jnp.int32).reshape(NBLK, 1, BN)
    ea = edge_attr.astype(jnp.int32)
    packed = jnp.stack([ea[:, 0], ea[:, 1], ea[:, 2]])
    packed = jnp.pad(packed, ((0, 5), (0, 0)))
    packed = packed.reshape(8, NC * NS, EPT).transpose(1, 0, 2)
    esh = (NC, NS, NSB, SB, C)
    src3 = edge_index[0].astype(jnp.int32).reshape(esh)
    dst4 = edge_index[1].astype(jnp.int32).reshape(esh)
    emb_pad = jnp.pad(p['atom_emb'], ((0, 0), (0, 28), (0, 0)))

    ce3, cc3 = _k_prep(packed)
    ce3 = ce3.reshape(esh)
    cc3 = cc3.reshape(esh)

    comb_all = _k_comb(p['bond_emb']).reshape(3, TAB * EMB)

    hin0, pool0 = _k_atom(xt, batch3, emb_pad, p['vn_emb'])
    vn0 = jnp.broadcast_to(p['vn_emb'][0][None, :], (G, EMB))

    def vn_step(layer, pool, vn_prev):
        return _k_vn(pool, vn_prev,
                     p['vn_W1'][layer], p['vn_b1'][layer], p['vn_g1'][layer],
                     p['vn_be1'][layer], p['vn_W2'][layer], p['vn_b2'][layer],
                     p['vn_g2'][layer], p['vn_be2'][layer])

    def mlp_args(layer):
        return (p['eps'][layer].reshape(1, 1), p['mlp_W1'][layer],
                p['mlp_b1'][layer], p['mlp_g1'][layer], p['mlp_be1'][layer],
                p['mlp_W2'][layer], p['mlp_b2'][layer], p['bn_g'][layer],
                p['bn_b'][layer])

    vn1 = vn_step(0, pool0, vn0)
    ag = _edge_kernel(hin0, src3, ce3, cc3, dst4, comb_all[0])
    hin1, pool1 = _k_mlp(0, hin0, ag[0], ag[1], batch3, vn1, *mlp_args(0))

    vn2 = vn_step(1, pool1, vn1)
    ag = _edge_kernel(hin1, src3, ce3, cc3, dst4, comb_all[1])
    hin2, _ = _k_mlp(1, hin1, ag[0], ag[1], batch3, vn2, *mlp_args(1))

    ag = _edge_kernel(hin2, src3, ce3, cc3, dst4, comb_all[2])
    h_out, _ = _k_mlp(2, hin2, ag[0], ag[1], batch3, vn2, *mlp_args(2))

    return h_out, jnp.stack([vn1, vn2], axis=1)
